# Initial kernel scaffold; baseline (speedup 1.0000x reference)
import jax
import jax.numpy as jnp
from jax.experimental import pallas as pl
from jax.experimental.pallas import tpu as pltpu

_B, _C, _H, _W = 8, 384, 14, 14
_E, _K, _R = 8, 2, 4
_HID = _R * _C
_T = _B * _H * _W
_CAP = int(1.25 * _T * _K / _E)


def _conv_ln_body(xp_ref, w_ref, dwb_ref, lnw_ref, lnb_ref, out_ref):
    xp = xp_ref[...]
    acc = jnp.zeros((_B, _H, _W, _C), jnp.float32)
    for di in range(7):
        for dj in range(7):
            sl = jax.lax.slice(xp, (0, di, dj, 0), (_B, di + _H, dj + _W, _C))
            wrow = jax.lax.slice(w_ref[...], (di * 7 + dj, 0),
                                 (di * 7 + dj + 1, _C)).reshape(1, 1, 1, _C)
            acc = acc + sl * wrow
    acc = acc + dwb_ref[...].reshape(1, 1, 1, _C)
    mu = jnp.mean(acc, axis=-1, keepdims=True)
    xc = acc - mu
    var = jnp.mean(xc * xc, axis=-1, keepdims=True)
    out_ref[...] = (xc * jax.lax.rsqrt(var + 1e-6)
                    * lnw_ref[...].reshape(1, 1, 1, _C)
                    + lnb_ref[...].reshape(1, 1, 1, _C))


def _router_body(x_ref, rw_ref, gates_ref):
    x = x_ref[...]
    logits = jnp.dot(x, rw_ref[...], preferred_element_type=jnp.float32)
    m = jnp.max(logits, axis=-1, keepdims=True)
    ex = jnp.exp(logits - m)
    probs = ex / jnp.sum(ex, axis=-1, keepdims=True)

    cols = [jax.lax.slice(probs, (0, e), (_T, e + 1)) for e in range(_E)]
    v1 = cols[0]
    i1 = jnp.zeros((_T, 1), jnp.int32)
    for e in range(1, _E):
        take = cols[e] > v1
        v1 = jnp.where(take, cols[e], v1)
        i1 = jnp.where(take, e, i1)
    v2 = jnp.full((_T, 1), -1.0, jnp.float32)
    i2 = jnp.zeros((_T, 1), jnp.int32)
    for e in range(_E):
        cand = jnp.where(i1 == e, -1.0, cols[e])
        take = cand > v2
        v2 = jnp.where(take, cand, v2)
        i2 = jnp.where(take, e, i2)

    s = v1 + v2
    wk1 = v1 / s
    wk2 = v2 / s

    ones_col = jnp.ones((_T, 1), jnp.float32)
    bp = jax.lax.dot_general(ones_col, v1, (((1,), (1,)), ((), ())),
                             preferred_element_type=jnp.float32)
    row_i = jax.lax.broadcasted_iota(jnp.int32, (_T, _T), 0)
    col_i = jax.lax.broadcasted_iota(jnp.int32, (_T, _T), 1)
    prec = ((bp > v1).astype(jnp.float32)
            + ((bp == v1) & (col_i < row_i)).astype(jnp.float32))

    lane = jax.lax.broadcasted_iota(jnp.int32, (_T, _E), 1)
    oh1 = (lane == i1).astype(jnp.float32)
    oh2 = (lane == i2).astype(jnp.float32)
    member = oh1 + oh2
    rank_all = jnp.dot(prec, member, preferred_element_type=jnp.float32)
    rank1 = jnp.sum(oh1 * rank_all, axis=-1, keepdims=True)
    rank2 = jnp.sum(oh2 * rank_all, axis=-1, keepdims=True)
    keep1 = (rank1 < _CAP).astype(jnp.float32)
    keep2 = (rank2 < _CAP).astype(jnp.float32)

    gates_ref[...] = oh1 * (wk1 * keep1) + oh2 * (wk2 * keep2)


def _gelu_exact(h):
    return 0.5 * h * (1.0 + jax.lax.erf(h * 0.7071067811865476))


def _expert_body(xb_ref, gates_ref, w1_ref, b1_ref, w2_ref, b2_ref, out_ref):
    e = pl.program_id(0)

    @pl.when(e == 0)
    def _():
        out_ref[...] = jnp.zeros_like(out_ref)

    h = jnp.dot(xb_ref[...], w1_ref[0],
                preferred_element_type=jnp.float32) + b1_ref[...]
    g = _gelu_exact(h)
    y = jnp.dot(g.astype(jnp.bfloat16), w2_ref[0],
                preferred_element_type=jnp.float32) + b2_ref[...]
    lane = jax.lax.broadcasted_iota(jnp.int32, (_T, _E), 1)
    gate = jnp.sum(jnp.where(lane == e, gates_ref[...], 0.0),
                   axis=-1, keepdims=True)
    out_ref[...] += gate * y


def kernel(input, dw_w, dw_b, ln_w, ln_b, router_w, w1, b1, w2, b2, layer_scale):
    x_nhwc = jnp.transpose(input, (0, 2, 3, 1))
    xp = jnp.pad(x_nhwc, ((0, 0), (3, 3), (3, 3), (0, 0)))
    wt = jnp.transpose(dw_w[:, 0], (1, 2, 0)).reshape(49, _C)

    ln = pl.pallas_call(
        _conv_ln_body,
        out_shape=jax.ShapeDtypeStruct((_B, _H, _W, _C), jnp.float32),
    )(xp, wt, dw_b.reshape(1, _C), ln_w.reshape(1, _C), ln_b.reshape(1, _C))

    x_flat = ln.reshape(_T, _C)
    gates = pl.pallas_call(
        _router_body,
        out_shape=jax.ShapeDtypeStruct((_T, _E), jnp.float32),
    )(x_flat, router_w)

    xb = x_flat.astype(jnp.bfloat16)
    moe_flat = pl.pallas_call(
        _expert_body,
        grid=(_E,),
        in_specs=[
            pl.BlockSpec((_T, _C), lambda e: (0, 0)),
            pl.BlockSpec((_T, _E), lambda e: (0, 0)),
            pl.BlockSpec((1, _C, _HID), lambda e: (e, 0, 0)),
            pl.BlockSpec((1, _HID), lambda e: (e, 0)),
            pl.BlockSpec((1, _HID, _C), lambda e: (e, 0, 0)),
            pl.BlockSpec((1, _C), lambda e: (e, 0)),
        ],
        out_specs=pl.BlockSpec((_T, _C), lambda e: (0, 0)),
        out_shape=jax.ShapeDtypeStruct((_T, _C), jnp.float32),
    )(xb, gates, w1.astype(jnp.bfloat16), b1, w2.astype(jnp.bfloat16), b2)

    moe = jnp.transpose(moe_flat.reshape(_B, _H, _W, _C), (0, 3, 1, 2))
    x_skip = jnp.transpose(ln, (0, 3, 1, 2))
    return input + x_skip + layer_scale[None] * moe


# speedup vs baseline: 2.6396x; 2.6396x over previous
import jax
import jax.numpy as jnp
from jax.experimental import pallas as pl
from jax.experimental.pallas import tpu as pltpu

_B, _C, _H, _W = 8, 384, 14, 14
_E, _K, _R = 8, 2, 4
_HID = _R * _C
_T = _B * _H * _W
_CAP = int(1.25 * _T * _K / _E)


def _conv_ln_body(xp_ref, w_ref, dwb_ref, lnw_ref, lnb_ref, out_ref):
    xp = xp_ref[...]
    acc = jnp.zeros((_B, _H, _W, _C), jnp.float32)
    for di in range(7):
        for dj in range(7):
            sl = jax.lax.slice(xp, (0, di, dj, 0), (_B, di + _H, dj + _W, _C))
            wrow = jax.lax.slice(w_ref[...], (di * 7 + dj, 0),
                                 (di * 7 + dj + 1, _C)).reshape(1, 1, 1, _C)
            acc = acc + sl * wrow
    acc = acc + dwb_ref[...].reshape(1, 1, 1, _C)
    mu = jnp.mean(acc, axis=-1, keepdims=True)
    xc = acc - mu
    var = jnp.mean(xc * xc, axis=-1, keepdims=True)
    out_ref[...] = (xc * jax.lax.rsqrt(var + 1e-6)
                    * lnw_ref[...].reshape(1, 1, 1, _C)
                    + lnb_ref[...].reshape(1, 1, 1, _C))


def _router_body(x_ref, rw_ref, gates_ref):
    x = x_ref[...]
    logits = jnp.dot(x, rw_ref[...], preferred_element_type=jnp.float32)
    m = jnp.max(logits, axis=-1, keepdims=True)
    ex = jnp.exp(logits - m)
    probs = ex / jnp.sum(ex, axis=-1, keepdims=True)

    cols = [jax.lax.slice(probs, (0, e), (_T, e + 1)) for e in range(_E)]
    v1 = cols[0]
    i1 = jnp.zeros((_T, 1), jnp.int32)
    for e in range(1, _E):
        take = cols[e] > v1
        v1 = jnp.where(take, cols[e], v1)
        i1 = jnp.where(take, e, i1)
    v2 = jnp.full((_T, 1), -1.0, jnp.float32)
    i2 = jnp.zeros((_T, 1), jnp.int32)
    for e in range(_E):
        cand = jnp.where(i1 == e, -1.0, cols[e])
        take = cand > v2
        v2 = jnp.where(take, cand, v2)
        i2 = jnp.where(take, e, i2)

    s = v1 + v2
    wk1 = v1 / s
    wk2 = v2 / s

    ones_col = jnp.ones((_T, 1), jnp.float32)
    bp = jax.lax.dot_general(ones_col, v1, (((1,), (1,)), ((), ())),
                             preferred_element_type=jnp.float32)
    row_i = jax.lax.broadcasted_iota(jnp.int32, (_T, _T), 0)
    col_i = jax.lax.broadcasted_iota(jnp.int32, (_T, _T), 1)
    prec = ((bp > v1).astype(jnp.float32)
            + ((bp == v1) & (col_i < row_i)).astype(jnp.float32))

    lane = jax.lax.broadcasted_iota(jnp.int32, (_T, _E), 1)
    oh1 = (lane == i1).astype(jnp.float32)
    oh2 = (lane == i2).astype(jnp.float32)
    member = oh1 + oh2
    rank_all = jnp.dot(prec, member, preferred_element_type=jnp.float32)
    rank1 = jnp.sum(oh1 * rank_all, axis=-1, keepdims=True)
    rank2 = jnp.sum(oh2 * rank_all, axis=-1, keepdims=True)
    keep1 = (rank1 < _CAP).astype(jnp.float32)
    keep2 = (rank2 < _CAP).astype(jnp.float32)

    gates_ref[...] = oh1 * (wk1 * keep1) + oh2 * (wk2 * keep2)


def _gelu_exact(h):
    return 0.5 * h * (1.0 + jax.lax.erf(h * 0.7071067811865476))


def _expert_body(xb_ref, gates_ref, w1_ref, b1_ref, w2_ref, b2_ref, out_ref):
    e = pl.program_id(0)

    @pl.when(e == 0)
    def _():
        out_ref[...] = jnp.zeros_like(out_ref)

    h = jnp.dot(xb_ref[...], w1_ref[0],
                preferred_element_type=jnp.float32) + b1_ref[0]
    g = _gelu_exact(h)
    y = jnp.dot(g.astype(jnp.bfloat16), w2_ref[0],
                preferred_element_type=jnp.float32) + b2_ref[0]
    lane = jax.lax.broadcasted_iota(jnp.int32, (_T, _E), 1)
    gate = jnp.sum(jnp.where(lane == e, gates_ref[...], 0.0),
                   axis=-1, keepdims=True)
    out_ref[...] += gate * y


def kernel(input, dw_w, dw_b, ln_w, ln_b, router_w, w1, b1, w2, b2, layer_scale):
    x_nhwc = jnp.transpose(input, (0, 2, 3, 1))
    xp = jnp.pad(x_nhwc, ((0, 0), (3, 3), (3, 3), (0, 0)))
    wt = jnp.transpose(dw_w[:, 0], (1, 2, 0)).reshape(49, _C)

    ln = pl.pallas_call(
        _conv_ln_body,
        out_shape=jax.ShapeDtypeStruct((_B, _H, _W, _C), jnp.float32),
    )(xp, wt, dw_b.reshape(1, _C), ln_w.reshape(1, _C), ln_b.reshape(1, _C))

    x_flat = ln.reshape(_T, _C)
    gates = pl.pallas_call(
        _router_body,
        out_shape=jax.ShapeDtypeStruct((_T, _E), jnp.float32),
    )(x_flat, router_w)

    xb = x_flat.astype(jnp.bfloat16)
    moe_flat = pl.pallas_call(
        _expert_body,
        grid=(_E,),
        in_specs=[
            pl.BlockSpec((_T, _C), lambda e: (0, 0)),
            pl.BlockSpec((_T, _E), lambda e: (0, 0)),
            pl.BlockSpec((1, _C, _HID), lambda e: (e, 0, 0)),
            pl.BlockSpec((1, 1, _HID), lambda e: (e, 0, 0)),
            pl.BlockSpec((1, _HID, _C), lambda e: (e, 0, 0)),
            pl.BlockSpec((1, 1, _C), lambda e: (e, 0, 0)),
        ],
        out_specs=pl.BlockSpec((_T, _C), lambda e: (0, 0)),
        out_shape=jax.ShapeDtypeStruct((_T, _C), jnp.float32),
    )(xb, gates, w1.astype(jnp.bfloat16), b1.reshape(_E, 1, _HID),
      w2.astype(jnp.bfloat16), b2.reshape(_E, 1, _C))

    moe = jnp.transpose(moe_flat.reshape(_B, _H, _W, _C), (0, 3, 1, 2))
    x_skip = jnp.transpose(ln, (0, 3, 1, 2))
    return input + x_skip + layer_scale[None] * moe
